# baseline (speedup 1.0000x reference)
import functools

import jax
import jax.numpy as jnp
from jax import lax
from jax.experimental import pallas as pl
from jax.experimental.pallas import tpu as pltpu
from jax.experimental.pallas import tpu_sc as plsc

F32 = jnp.float32
BM = 256
BF = 512
GMAX = 24
EP = 128


def _ln(x, g, b, eps=1e-5):
    m = jnp.mean(x, axis=-1, keepdims=True)
    v = jnp.mean((x - m) * (x - m), axis=-1, keepdims=True)
    return (x - m) * jax.lax.rsqrt(v + eps) * g + b


def _qkv_body(x_ref, w_ref, b_ref, g_ref, bb_ref, o_ref):
    h = _ln(x_ref[...], g_ref[...], bb_ref[...])
    o_ref[...] = jnp.dot(h, w_ref[...], preferred_element_type=F32,
                         precision=jax.lax.Precision.HIGHEST) + b_ref[...]


def _qkv(x, w, b, g, bb, BN=1024):
    S, D = x.shape
    N = w.shape[1]
    return pl.pallas_call(
        _qkv_body,
        grid=(S // BM, N // BN),
        in_specs=[
            pl.BlockSpec((BM, D), lambda i, j: (i, 0)),
            pl.BlockSpec((D, BN), lambda i, j: (0, j)),
            pl.BlockSpec((1, BN), lambda i, j: (0, j)),
            pl.BlockSpec((1, D), lambda i, j: (0, 0)),
            pl.BlockSpec((1, D), lambda i, j: (0, 0)),
        ],
        out_specs=pl.BlockSpec((BM, BN), lambda i, j: (i, j)),
        out_shape=jax.ShapeDtypeStruct((S, N), F32),
    )(x, w, b, g, bb)


def _attn_body(q_ref, k_ref, v_ref, o_ref, *, scale):
    q = q_ref[0] * scale
    s = jax.lax.dot_general(q, k_ref[0], (((1,), (1,)), ((), ())),
                            preferred_element_type=F32,
                            precision=jax.lax.Precision.HIGHEST)
    m = jnp.max(s, axis=-1, keepdims=True)
    p = jnp.exp(s - m)
    p = p / jnp.sum(p, axis=-1, keepdims=True)
    o_ref[0] = jnp.dot(p, v_ref[0], preferred_element_type=F32,
                       precision=jax.lax.Precision.HIGHEST)


def _attn(q3, k3, v3, BQ=512):
    H, S, HD = q3.shape
    body = functools.partial(_attn_body, scale=HD ** -0.5)
    return pl.pallas_call(
        body,
        grid=(H, S // BQ),
        in_specs=[
            pl.BlockSpec((1, BQ, HD), lambda h, i: (h, i, 0)),
            pl.BlockSpec((1, S, HD), lambda h, i: (h, 0, 0)),
            pl.BlockSpec((1, S, HD), lambda h, i: (h, 0, 0)),
        ],
        out_specs=pl.BlockSpec((1, BQ, HD), lambda h, i: (h, i, 0)),
        out_shape=jax.ShapeDtypeStruct((H, S, HD), F32),
    )(q3, k3, v3)


def _oproj_body(ctx_ref, ow_ref, ob_ref, res_ref, g2_ref, b2_ref, gate_ref,
                h_ref, h2_ref, lg_ref, e12_ref, w12_ref, *, E):
    h = (jnp.dot(ctx_ref[...], ow_ref[...], preferred_element_type=F32,
                 precision=jax.lax.Precision.HIGHEST)
         + ob_ref[...] + res_ref[...])
    h_ref[...] = h
    h2 = _ln(h, g2_ref[...], b2_ref[...])
    h2_ref[...] = h2
    lg = jax.lax.dot_general(h2, gate_ref[...], (((1,), (1,)), ((), ())),
                             preferred_element_type=F32,
                             precision=jax.lax.Precision.HIGHEST)
    lg_ref[...] = lg

    lane = jax.lax.broadcasted_iota(jnp.int32, lg.shape, 1)
    lgm = jnp.where(lane < E, lg, -jnp.inf)
    mx = jnp.max(lgm, axis=-1, keepdims=True)
    p = jnp.exp(lgm - mx)
    rw = p / jnp.sum(p, axis=-1, keepdims=True)
    m1 = jnp.max(rw, axis=-1, keepdims=True)
    i1 = jnp.min(jnp.where(rw == m1, lane, 9999), axis=-1, keepdims=True)
    rw2 = jnp.where(lane == i1, -jnp.inf, rw)
    m2 = jnp.max(rw2, axis=-1, keepdims=True)
    i2 = jnp.min(jnp.where(rw2 == m2, lane, 9999), axis=-1, keepdims=True)
    tot = m1 + m2
    e12_ref[0] = jnp.broadcast_to(i1, lg.shape)
    e12_ref[1] = jnp.broadcast_to(i2, lg.shape)
    w12_ref[0] = jnp.broadcast_to(m1 / tot, lg.shape)
    w12_ref[1] = jnp.broadcast_to(m2 / tot, lg.shape)


def _oproj(ctx, ow, ob, res, g2, b2, gate_pad, E):
    S, D = ctx.shape
    body = functools.partial(_oproj_body, E=E)
    return pl.pallas_call(
        body,
        grid=(S // BM,),
        in_specs=[
            pl.BlockSpec((BM, D), lambda i: (i, 0)),
            pl.BlockSpec((D, D), lambda i: (0, 0)),
            pl.BlockSpec((1, D), lambda i: (0, 0)),
            pl.BlockSpec((BM, D), lambda i: (i, 0)),
            pl.BlockSpec((1, D), lambda i: (0, 0)),
            pl.BlockSpec((1, D), lambda i: (0, 0)),
            pl.BlockSpec((EP, D), lambda i: (0, 0)),
        ],
        out_specs=[
            pl.BlockSpec((BM, D), lambda i: (i, 0)),
            pl.BlockSpec((BM, D), lambda i: (i, 0)),
            pl.BlockSpec((BM, EP), lambda i: (i, 0)),
            pl.BlockSpec((2, BM, EP), lambda i: (0, i, 0)),
            pl.BlockSpec((2, BM, EP), lambda i: (0, i, 0)),
        ],
        out_shape=[
            jax.ShapeDtypeStruct((S, D), F32),
            jax.ShapeDtypeStruct((S, D), F32),
            jax.ShapeDtypeStruct((S, EP), F32),
            jax.ShapeDtypeStruct((2, S, EP), jnp.int32),
            jax.ShapeDtypeStruct((2, S, EP), F32),
        ],
    )(ctx, ow, ob, res, g2, b2, gate_pad)


def _pos_body(e_ref, pos_ref, cnt_ref, eflat_ref, carry):
    k = pl.program_id(0)
    m = pl.program_id(1)

    @pl.when((k == 0) & (m == 0))
    def _():
        carry[...] = jnp.zeros_like(carry)

    e = e_ref[0][:, :1]
    eflat_ref[...] = e
    lane = jax.lax.broadcasted_iota(jnp.int32, (BM, EP), 1)
    mask = (e == lane).astype(F32)
    r = jax.lax.broadcasted_iota(jnp.int32, (BM, BM), 0)
    c = jax.lax.broadcasted_iota(jnp.int32, (BM, BM), 1)
    tri = (c < r).astype(F32)
    prefix = jnp.dot(tri, mask, preferred_element_type=F32)
    poswithin = jnp.sum(prefix * mask, axis=-1, keepdims=True)
    carried = jnp.sum(mask * carry[...], axis=-1, keepdims=True)
    pos_ref[...] = poswithin + carried
    carry[...] += jnp.sum(mask, axis=0, keepdims=True)
    cnt_ref[...] = carry[...]


def _dispatch_pos(e12, S):
    NM = S // BM
    return pl.pallas_call(
        _pos_body,
        grid=(2, NM),
        in_specs=[pl.BlockSpec((1, BM, EP), lambda k, m: (k, m, 0))],
        out_specs=[
            pl.BlockSpec((BM, 1), lambda k, m: (k * NM + m, 0)),
            pl.BlockSpec((1, EP), lambda k, m: (0, 0)),
            pl.BlockSpec((BM, 1), lambda k, m: (k * NM + m, 0)),
        ],
        out_shape=[
            jax.ShapeDtypeStruct((2 * S, 1), F32),
            jax.ShapeDtypeStruct((1, EP), F32),
            jax.ShapeDtypeStruct((2 * S, 1), jnp.int32),
        ],
        scratch_shapes=[pltpu.VMEM((1, EP), F32)],
    )(e12)


def _dest_body(pos_ref, eflat_ref, cnt_ref, dest_ref, bexp_ref, *, E):
    cnt = cnt_ref[...]
    pc = jnp.ceil(cnt / BM) * BM
    ge = jax.lax.broadcasted_iota(jnp.int32, (EP, EP), 0)
    gl = jax.lax.broadcasted_iota(jnp.int32, (EP, EP), 1)
    tri = (ge > gl).astype(F32)
    offrow = jax.lax.dot_general(
        pc, tri, (((1,), (1,)), ((), ())),
        preferred_element_type=F32)
    e = eflat_ref[...]
    lane = jax.lax.broadcasted_iota(jnp.int32, (e.shape[0], EP), 1)
    onehot = (e == lane).astype(F32)
    offsel = jnp.sum(onehot * offrow, axis=-1, keepdims=True)
    dest_ref[...] = (offsel + pos_ref[...]).astype(jnp.int32)
    gidx = jax.lax.broadcasted_iota(jnp.int32, (EP, EP), 0)
    elane = jax.lax.broadcasted_iota(jnp.int32, (EP, EP), 1)
    ind = ((offrow <= gidx.astype(F32) * BM) & (elane < E)).astype(jnp.int32)
    be = jnp.sum(ind, axis=-1, keepdims=True) - 1
    nblk = (jnp.sum(pc) / BM).astype(jnp.int32)
    rowi = jax.lax.broadcasted_iota(jnp.int32, (EP, 1), 0)
    bexp_ref[...] = jnp.where(rowi == GMAX, nblk, be)


def _dispatch_dest(pos, eflat, cnt, E):
    A = pos.shape[0]
    return pl.pallas_call(
        functools.partial(_dest_body, E=E),
        grid=(1,),
        in_specs=[
            pl.BlockSpec((A, 1), lambda i: (0, 0)),
            pl.BlockSpec((A, 1), lambda i: (0, 0)),
            pl.BlockSpec((1, EP), lambda i: (0, 0)),
        ],
        out_specs=[
            pl.BlockSpec((A, 1), lambda i: (0, 0)),
            pl.BlockSpec((EP, 1), lambda i: (0, 0)),
        ],
        out_shape=[
            jax.ShapeDtypeStruct((A, 1), jnp.int32),
            jax.ShapeDtypeStruct((EP, 1), jnp.int32),
        ],
    )(pos, eflat, cnt)


def _invert_body(dest_ref, tokhl_ref, tsrc_ref, *, S):
    g = pl.program_id(0)
    d = dest_ref[...]
    slot = (g * BM
            + jax.lax.broadcasted_iota(jnp.int32, (d.shape[0], BM), 1))
    onehot = (d == slot).astype(F32)
    t = jax.lax.dot_general(onehot, tokhl_ref[...], (((0,), (0,)), ((), ())),
                            preferred_element_type=F32)
    tok = t[:, :1] * 64.0 + t[:, 1:2]
    matched = t[:, 2:3]
    scol = (g * BM + jax.lax.broadcasted_iota(jnp.int32, (BM, 1), 0)) % S
    tsrc_ref[...] = (tok + (1.0 - matched) * scol.astype(F32)).astype(
        jnp.int32)


def _invert(dest, tokhl, S):
    A = dest.shape[0]
    return pl.pallas_call(
        functools.partial(_invert_body, S=S),
        grid=(GMAX,),
        in_specs=[
            pl.BlockSpec((A, 1), lambda g: (0, 0)),
            pl.BlockSpec((A, 3), lambda g: (0, 0)),
        ],
        out_specs=pl.BlockSpec((BM, 1), lambda g: (g, 0)),
        out_shape=jax.ShapeDtypeStruct((GMAX * BM, 1), jnp.int32),
    )(dest, tokhl)


def _sc_dispatch(h2, tsrc):
    S, D = h2.shape
    R = tsrc.shape[0]
    info = plsc.get_sparse_core_info()
    NC, NS = info.num_cores, info.num_subcores
    per_w = R // (NC * NS)
    CH = 64

    @functools.partial(
        pl.kernel,
        out_type=jax.ShapeDtypeStruct((R, D), F32),
        mesh=plsc.VectorSubcoreMesh(core_axis_name="c", subcore_axis_name="s"),
        scratch_types=[
            pltpu.VMEM((CH,), jnp.int32),
            pltpu.VMEM((CH, D), F32),
            pltpu.SemaphoreType.DMA,
        ],
    )
    def disp(h2_hbm, tsrc_hbm, xs_hbm, idx_v, rows_v, sem):
        wid = lax.axis_index("s") * NC + lax.axis_index("c")
        base = wid * per_w
        for c in range(per_w // CH):
            pltpu.sync_copy(tsrc_hbm.at[pl.ds(base + c * CH, CH)], idx_v)
            pltpu.async_copy(h2_hbm.at[idx_v], rows_v, sem).wait()
            pltpu.sync_copy(rows_v, xs_hbm.at[pl.ds(base + c * CH, CH)])

    return disp(h2, tsrc)


def _sc_combine_gather(Y, dest):
    R, D = Y.shape
    A = dest.shape[0]
    info = plsc.get_sparse_core_info()
    NC, NS = info.num_cores, info.num_subcores
    per_w = A // (NC * NS)
    CH = 64

    @functools.partial(
        pl.kernel,
        out_type=jax.ShapeDtypeStruct((A, D), F32),
        mesh=plsc.VectorSubcoreMesh(core_axis_name="c", subcore_axis_name="s"),
        scratch_types=[
            pltpu.VMEM((CH,), jnp.int32),
            pltpu.VMEM((CH, D), F32),
            pltpu.SemaphoreType.DMA,
        ],
    )
    def comb(y_hbm, dest_hbm, yy_hbm, idx_v, rows_v, sem):
        wid = lax.axis_index("s") * NC + lax.axis_index("c")
        base = wid * per_w
        for c in range(per_w // CH):
            pltpu.sync_copy(dest_hbm.at[pl.ds(base + c * CH, CH)], idx_v)
            pltpu.async_copy(y_hbm.at[idx_v], rows_v, sem).wait()
            pltpu.sync_copy(rows_v, yy_hbm.at[pl.ds(base + c * CH, CH)])

    return comb(Y, dest)


def _group_body(s_ref, xs_ref, fc1_ref, fc2_ref, o_ref):
    g = pl.program_id(0)

    @pl.when(g < s_ref[GMAX])
    def _():
        x = xs_ref[...].astype(jnp.bfloat16)
        t = jax.lax.dot_general(x, fc1_ref[0], (((1,), (1,)), ((), ())),
                                preferred_element_type=F32)
        t = 0.5 * t * (1.0 + jax.lax.erf(t * 0.7071067811865476))
        o_ref[...] = jax.lax.dot_general(
            t.astype(jnp.bfloat16), fc2_ref[0], (((1,), (1,)), ((), ())),
            preferred_element_type=F32)


def _moe_grouped(xs, fc1, fc2, bexp):
    R, D = xs.shape
    E, FF, _ = fc1.shape
    grid_spec = pltpu.PrefetchScalarGridSpec(
        num_scalar_prefetch=1,
        grid=(GMAX,),
        in_specs=[
            pl.BlockSpec((BM, D), lambda g, s: (g, 0)),
            pl.BlockSpec((1, FF, D),
                         lambda g, s: (jnp.clip(s[g], 0, E - 1), 0, 0)),
            pl.BlockSpec((1, D, FF),
                         lambda g, s: (jnp.clip(s[g], 0, E - 1), 0, 0)),
        ],
        out_specs=pl.BlockSpec((BM, D), lambda g, s: (g, 0)),
    )
    return pl.pallas_call(
        _group_body,
        grid_spec=grid_spec,
        out_shape=jax.ShapeDtypeStruct((GMAX * BM, D), F32),
    )(bexp, xs, fc1, fc2)


def _combine_body(h_ref, y0_ref, y1_ref, w12_ref, o_ref):
    w1 = w12_ref[0][:, :1]
    w2 = w12_ref[1][:, :1]
    o_ref[...] = h_ref[...] + w1 * y0_ref[...] + w2 * y1_ref[...]


def _combine(h, yy, w12):
    S, D = h.shape
    NM = S // BM
    return pl.pallas_call(
        _combine_body,
        grid=(NM,),
        in_specs=[
            pl.BlockSpec((BM, D), lambda m: (m, 0)),
            pl.BlockSpec((BM, D), lambda m: (m, 0)),
            pl.BlockSpec((BM, D), lambda m: (m + NM, 0)),
            pl.BlockSpec((2, BM, EP), lambda m: (0, m, 0)),
        ],
        out_specs=pl.BlockSpec((BM, D), lambda m: (m, 0)),
        out_shape=jax.ShapeDtypeStruct((S, D), F32),
    )(h, yy, yy, w12)


def kernel(hidden_states, ln1_g, ln1_b, q_w, q_b, k_w, k_b, v_w, v_b,
           o_w, o_b, ln2_g, ln2_b, gate_w, fc1_w, fc2_w):
    B, S, D = hidden_states.shape
    E = gate_w.shape[0]
    H = 16
    HD = D // H
    x = hidden_states.reshape(S, D)

    wqkv = jnp.concatenate([q_w.T, k_w.T, v_w.T], axis=1)
    bqkv = jnp.concatenate([q_b, k_b, v_b]).reshape(1, 3 * D)
    qkv = _qkv(x, wqkv, bqkv, ln1_g.reshape(1, D), ln1_b.reshape(1, D))

    def heads(t):
        return t.reshape(S, H, HD).transpose(1, 0, 2)

    q3 = heads(qkv[:, :D])
    k3 = heads(qkv[:, D:2 * D])
    v3 = heads(qkv[:, 2 * D:])
    ctx = _attn(q3, k3, v3).transpose(1, 0, 2).reshape(S, D)

    gate_pad = jnp.zeros((EP, D), F32).at[:E].set(gate_w)
    h, h2, lg, e12, w12 = _oproj(ctx, o_w.T, o_b.reshape(1, D), x,
                                 ln2_g.reshape(1, D), ln2_b.reshape(1, D),
                                 gate_pad, E)

    pos, cnt, eflat = _dispatch_pos(e12, S)
    dest, bexp = _dispatch_dest(pos, eflat, cnt, E)
    bexp_s = bexp.reshape(EP)[:GMAX + 1]
    tok = jnp.arange(2 * S, dtype=jnp.int32) % S
    tokhl = jnp.stack([tok // 64, tok % 64, jnp.ones_like(tok)],
                      axis=1).astype(F32)
    tsrc = _invert(dest, tokhl, S)

    xs = _sc_dispatch(h2, tsrc.reshape(-1))
    Y = _moe_grouped(xs, fc1_w.astype(jnp.bfloat16),
                     fc2_w.astype(jnp.bfloat16), bexp_s)
    yy = _sc_combine_gather(Y, dest.reshape(-1))
    out = _combine(h, yy, w12)
    return (out.reshape(B, S, D), lg[:, :E])


# speedup vs baseline: 2.0881x; 2.0881x over previous
import functools

import jax
import jax.numpy as jnp
from jax import lax
from jax.experimental import pallas as pl
from jax.experimental.pallas import tpu as pltpu
from jax.experimental.pallas import tpu_sc as plsc

F32 = jnp.float32
BM = 256
BF = 512
GMAX = 24
EP = 128


def _ln(x, g, b, eps=1e-5):
    m = jnp.mean(x, axis=-1, keepdims=True)
    v = jnp.mean((x - m) * (x - m), axis=-1, keepdims=True)
    return (x - m) * jax.lax.rsqrt(v + eps) * g + b


def _qkv_body(x_ref, w_ref, b_ref, g_ref, bb_ref, o_ref):
    h = _ln(x_ref[...], g_ref[...], bb_ref[...])
    o_ref[...] = jnp.dot(h, w_ref[...], preferred_element_type=F32) + b_ref[...]


def _qkv(x, w, b, g, bb, BN=1024):
    S, D = x.shape
    N = w.shape[1]
    return pl.pallas_call(
        _qkv_body,
        grid=(N // BN, S // BM),
        in_specs=[
            pl.BlockSpec((BM, D), lambda j, i: (i, 0)),
            pl.BlockSpec((D, BN), lambda j, i: (0, j)),
            pl.BlockSpec((1, BN), lambda j, i: (0, j)),
            pl.BlockSpec((1, D), lambda j, i: (0, 0)),
            pl.BlockSpec((1, D), lambda j, i: (0, 0)),
        ],
        out_specs=pl.BlockSpec((BM, BN), lambda j, i: (i, j)),
        out_shape=jax.ShapeDtypeStruct((S, N), F32),
    )(x, w, b, g, bb)


def _attn_body(q_ref, k_ref, v_ref, o_ref, *, scale):
    q = q_ref[0] * scale
    s = jax.lax.dot_general(q, k_ref[0], (((1,), (1,)), ((), ())),
                            preferred_element_type=F32)
    m = jnp.max(s, axis=-1, keepdims=True)
    p = jnp.exp(s - m)
    p = p / jnp.sum(p, axis=-1, keepdims=True)
    o_ref[0] = jnp.dot(p, v_ref[0], preferred_element_type=F32)


def _attn(q3, k3, v3, BQ=512):
    H, S, HD = q3.shape
    body = functools.partial(_attn_body, scale=HD ** -0.5)
    return pl.pallas_call(
        body,
        grid=(H, S // BQ),
        in_specs=[
            pl.BlockSpec((1, BQ, HD), lambda h, i: (h, i, 0)),
            pl.BlockSpec((1, S, HD), lambda h, i: (h, 0, 0)),
            pl.BlockSpec((1, S, HD), lambda h, i: (h, 0, 0)),
        ],
        out_specs=pl.BlockSpec((1, BQ, HD), lambda h, i: (h, i, 0)),
        out_shape=jax.ShapeDtypeStruct((H, S, HD), F32),
    )(q3, k3, v3)


def _oproj_body(ctx_ref, ow_ref, ob_ref, res_ref, g2_ref, b2_ref, gate_ref,
                h_ref, h2_ref, lg_ref, e12_ref, w12_ref, *, E):
    h = (jnp.dot(ctx_ref[...], ow_ref[...], preferred_element_type=F32)
         + ob_ref[...] + res_ref[...])
    h_ref[...] = h
    h2 = _ln(h, g2_ref[...], b2_ref[...])
    h2_ref[...] = h2
    lg = jax.lax.dot_general(h2, gate_ref[...], (((1,), (1,)), ((), ())),
                             preferred_element_type=F32)
    lg_ref[...] = lg

    lane = jax.lax.broadcasted_iota(jnp.int32, lg.shape, 1)
    lgm = jnp.where(lane < E, lg, -jnp.inf)
    mx = jnp.max(lgm, axis=-1, keepdims=True)
    p = jnp.exp(lgm - mx)
    rw = p / jnp.sum(p, axis=-1, keepdims=True)
    m1 = jnp.max(rw, axis=-1, keepdims=True)
    i1 = jnp.min(jnp.where(rw == m1, lane, 9999), axis=-1, keepdims=True)
    rw2 = jnp.where(lane == i1, -jnp.inf, rw)
    m2 = jnp.max(rw2, axis=-1, keepdims=True)
    i2 = jnp.min(jnp.where(rw2 == m2, lane, 9999), axis=-1, keepdims=True)
    tot = m1 + m2
    e12_ref[0] = jnp.broadcast_to(i1, lg.shape)
    e12_ref[1] = jnp.broadcast_to(i2, lg.shape)
    w12_ref[0] = jnp.broadcast_to(m1 / tot, lg.shape)
    w12_ref[1] = jnp.broadcast_to(m2 / tot, lg.shape)


def _oproj(ctx, ow, ob, res, g2, b2, gate_pad, E):
    S, D = ctx.shape
    body = functools.partial(_oproj_body, E=E)
    return pl.pallas_call(
        body,
        grid=(S // BM,),
        in_specs=[
            pl.BlockSpec((BM, D), lambda i: (i, 0)),
            pl.BlockSpec((D, D), lambda i: (0, 0)),
            pl.BlockSpec((1, D), lambda i: (0, 0)),
            pl.BlockSpec((BM, D), lambda i: (i, 0)),
            pl.BlockSpec((1, D), lambda i: (0, 0)),
            pl.BlockSpec((1, D), lambda i: (0, 0)),
            pl.BlockSpec((EP, D), lambda i: (0, 0)),
        ],
        out_specs=[
            pl.BlockSpec((BM, D), lambda i: (i, 0)),
            pl.BlockSpec((BM, D), lambda i: (i, 0)),
            pl.BlockSpec((BM, EP), lambda i: (i, 0)),
            pl.BlockSpec((2, BM, EP), lambda i: (0, i, 0)),
            pl.BlockSpec((2, BM, EP), lambda i: (0, i, 0)),
        ],
        out_shape=[
            jax.ShapeDtypeStruct((S, D), F32),
            jax.ShapeDtypeStruct((S, D), F32),
            jax.ShapeDtypeStruct((S, EP), F32),
            jax.ShapeDtypeStruct((2, S, EP), jnp.int32),
            jax.ShapeDtypeStruct((2, S, EP), F32),
        ],
    )(ctx, ow, ob, res, g2, b2, gate_pad)


def _pos_body(e_ref, pos_ref, cnt_ref, eflat_ref, carry):
    k = pl.program_id(0)
    m = pl.program_id(1)

    @pl.when((k == 0) & (m == 0))
    def _():
        carry[...] = jnp.zeros_like(carry)

    e = e_ref[0][:, :1]
    eflat_ref[...] = e
    lane = jax.lax.broadcasted_iota(jnp.int32, (BM, EP), 1)
    mask = (e == lane).astype(F32)
    r = jax.lax.broadcasted_iota(jnp.int32, (BM, BM), 0)
    c = jax.lax.broadcasted_iota(jnp.int32, (BM, BM), 1)
    tri = (c < r).astype(F32)
    prefix = jnp.dot(tri, mask, preferred_element_type=F32)
    poswithin = jnp.sum(prefix * mask, axis=-1, keepdims=True)
    carried = jnp.sum(mask * carry[...], axis=-1, keepdims=True)
    pos_ref[...] = poswithin + carried
    carry[...] += jnp.sum(mask, axis=0, keepdims=True)
    cnt_ref[...] = carry[...]


def _dispatch_pos(e12, S):
    NM = S // BM
    return pl.pallas_call(
        _pos_body,
        grid=(2, NM),
        in_specs=[pl.BlockSpec((1, BM, EP), lambda k, m: (k, m, 0))],
        out_specs=[
            pl.BlockSpec((BM, 1), lambda k, m: (k * NM + m, 0)),
            pl.BlockSpec((1, EP), lambda k, m: (0, 0)),
            pl.BlockSpec((BM, 1), lambda k, m: (k * NM + m, 0)),
        ],
        out_shape=[
            jax.ShapeDtypeStruct((2 * S, 1), F32),
            jax.ShapeDtypeStruct((1, EP), F32),
            jax.ShapeDtypeStruct((2 * S, 1), jnp.int32),
        ],
        scratch_shapes=[pltpu.VMEM((1, EP), F32)],
    )(e12)


def _dest_body(pos_ref, eflat_ref, cnt_ref, dest_ref, bexp_ref, *, E):
    cnt = cnt_ref[...]
    pc = jnp.ceil(cnt / BM) * BM
    ge = jax.lax.broadcasted_iota(jnp.int32, (EP, EP), 0)
    gl = jax.lax.broadcasted_iota(jnp.int32, (EP, EP), 1)
    tri = (ge > gl).astype(F32)
    offrow = jax.lax.dot_general(
        pc, tri, (((1,), (1,)), ((), ())),
        preferred_element_type=F32)
    e = eflat_ref[...]
    lane = jax.lax.broadcasted_iota(jnp.int32, (e.shape[0], EP), 1)
    onehot = (e == lane).astype(F32)
    offsel = jnp.sum(onehot * offrow, axis=-1, keepdims=True)
    dest_ref[...] = (offsel + pos_ref[...]).astype(jnp.int32)
    gidx = jax.lax.broadcasted_iota(jnp.int32, (EP, EP), 0)
    elane = jax.lax.broadcasted_iota(jnp.int32, (EP, EP), 1)
    ind = ((offrow <= gidx.astype(F32) * BM) & (elane < E)).astype(jnp.int32)
    be = jnp.sum(ind, axis=-1, keepdims=True) - 1
    nblk = (jnp.sum(pc) / BM).astype(jnp.int32)
    rowi = jax.lax.broadcasted_iota(jnp.int32, (EP, 1), 0)
    bexp_ref[...] = jnp.where(rowi == GMAX, nblk, be)


def _dispatch_dest(pos, eflat, cnt, E):
    A = pos.shape[0]
    return pl.pallas_call(
        functools.partial(_dest_body, E=E),
        grid=(1,),
        in_specs=[
            pl.BlockSpec((A, 1), lambda i: (0, 0)),
            pl.BlockSpec((A, 1), lambda i: (0, 0)),
            pl.BlockSpec((1, EP), lambda i: (0, 0)),
        ],
        out_specs=[
            pl.BlockSpec((A, 1), lambda i: (0, 0)),
            pl.BlockSpec((EP, 1), lambda i: (0, 0)),
        ],
        out_shape=[
            jax.ShapeDtypeStruct((A, 1), jnp.int32),
            jax.ShapeDtypeStruct((EP, 1), jnp.int32),
        ],
    )(pos, eflat, cnt)


def _invert_body(dest_ref, tokhl_ref, tsrc_ref, *, S):
    g = pl.program_id(0)
    d = dest_ref[...]
    slot = (g * BM
            + jax.lax.broadcasted_iota(jnp.int32, (d.shape[0], BM), 1))
    onehot = (d == slot).astype(F32)
    t = jax.lax.dot_general(onehot, tokhl_ref[...], (((0,), (0,)), ((), ())),
                            preferred_element_type=F32)
    tok = t[:, :1] * 64.0 + t[:, 1:2]
    matched = t[:, 2:3]
    scol = (g * BM + jax.lax.broadcasted_iota(jnp.int32, (BM, 1), 0)) % S
    tsrc_ref[...] = (tok + (1.0 - matched) * scol.astype(F32)).astype(
        jnp.int32)


def _invert(dest, tokhl, S):
    A = dest.shape[0]
    return pl.pallas_call(
        functools.partial(_invert_body, S=S),
        grid=(GMAX,),
        in_specs=[
            pl.BlockSpec((A, 1), lambda g: (0, 0)),
            pl.BlockSpec((A, 3), lambda g: (0, 0)),
        ],
        out_specs=pl.BlockSpec((BM, 1), lambda g: (g, 0)),
        out_shape=jax.ShapeDtypeStruct((GMAX * BM, 1), jnp.int32),
    )(dest, tokhl)


def _sc_dispatch(h2, tsrc):
    S, D = h2.shape
    R = tsrc.shape[0]
    info = plsc.get_sparse_core_info()
    NC, NS = info.num_cores, info.num_subcores
    per_w = R // (NC * NS)
    CH = 64

    @functools.partial(
        pl.kernel,
        out_type=jax.ShapeDtypeStruct((R, D), F32),
        mesh=plsc.VectorSubcoreMesh(core_axis_name="c", subcore_axis_name="s"),
        scratch_types=[
            pltpu.VMEM((CH,), jnp.int32),
            pltpu.VMEM((CH, D), F32),
            pltpu.SemaphoreType.DMA,
        ],
    )
    def disp(h2_hbm, tsrc_hbm, xs_hbm, idx_v, rows_v, sem):
        wid = lax.axis_index("s") * NC + lax.axis_index("c")
        base = wid * per_w
        for c in range(per_w // CH):
            pltpu.sync_copy(tsrc_hbm.at[pl.ds(base + c * CH, CH)], idx_v)
            pltpu.async_copy(h2_hbm.at[idx_v], rows_v, sem).wait()
            pltpu.sync_copy(rows_v, xs_hbm.at[pl.ds(base + c * CH, CH)])

    return disp(h2, tsrc)


def _sc_combine_gather(Y, dest):
    R, D = Y.shape
    A = dest.shape[0]
    info = plsc.get_sparse_core_info()
    NC, NS = info.num_cores, info.num_subcores
    per_w = A // (NC * NS)
    CH = 64

    @functools.partial(
        pl.kernel,
        out_type=jax.ShapeDtypeStruct((A, D), F32),
        mesh=plsc.VectorSubcoreMesh(core_axis_name="c", subcore_axis_name="s"),
        scratch_types=[
            pltpu.VMEM((CH,), jnp.int32),
            pltpu.VMEM((CH, D), F32),
            pltpu.SemaphoreType.DMA,
        ],
    )
    def comb(y_hbm, dest_hbm, yy_hbm, idx_v, rows_v, sem):
        wid = lax.axis_index("s") * NC + lax.axis_index("c")
        base = wid * per_w
        for c in range(per_w // CH):
            pltpu.sync_copy(dest_hbm.at[pl.ds(base + c * CH, CH)], idx_v)
            pltpu.async_copy(y_hbm.at[idx_v], rows_v, sem).wait()
            pltpu.sync_copy(rows_v, yy_hbm.at[pl.ds(base + c * CH, CH)])

    return comb(Y, dest)


def _group_body(s_ref, xs_ref, fc1_ref, fc2_ref, o_ref):
    g = pl.program_id(0)

    @pl.when(g < s_ref[GMAX])
    def _():
        x = xs_ref[...].astype(jnp.bfloat16)
        t = jax.lax.dot_general(x, fc1_ref[0], (((1,), (1,)), ((), ())),
                                preferred_element_type=F32)
        t = 0.5 * t * (1.0 + jax.lax.erf(t * 0.7071067811865476))
        o_ref[...] = jax.lax.dot_general(
            t.astype(jnp.bfloat16), fc2_ref[0], (((1,), (1,)), ((), ())),
            preferred_element_type=F32)


def _moe_grouped(xs, fc1, fc2, bexp):
    R, D = xs.shape
    E, FF, _ = fc1.shape
    grid_spec = pltpu.PrefetchScalarGridSpec(
        num_scalar_prefetch=1,
        grid=(GMAX,),
        in_specs=[
            pl.BlockSpec((BM, D), lambda g, s: (g, 0)),
            pl.BlockSpec((1, FF, D),
                         lambda g, s: (jnp.clip(s[g], 0, E - 1), 0, 0)),
            pl.BlockSpec((1, D, FF),
                         lambda g, s: (jnp.clip(s[g], 0, E - 1), 0, 0)),
        ],
        out_specs=pl.BlockSpec((BM, D), lambda g, s: (g, 0)),
    )
    return pl.pallas_call(
        _group_body,
        grid_spec=grid_spec,
        out_shape=jax.ShapeDtypeStruct((GMAX * BM, D), F32),
    )(bexp, xs, fc1, fc2)


def _combine_body(h_ref, y0_ref, y1_ref, w12_ref, o_ref):
    w1 = w12_ref[0][:, :1]
    w2 = w12_ref[1][:, :1]
    o_ref[...] = h_ref[...] + w1 * y0_ref[...] + w2 * y1_ref[...]


def _combine(h, yy, w12):
    S, D = h.shape
    NM = S // BM
    return pl.pallas_call(
        _combine_body,
        grid=(NM,),
        in_specs=[
            pl.BlockSpec((BM, D), lambda m: (m, 0)),
            pl.BlockSpec((BM, D), lambda m: (m, 0)),
            pl.BlockSpec((BM, D), lambda m: (m + NM, 0)),
            pl.BlockSpec((2, BM, EP), lambda m: (0, m, 0)),
        ],
        out_specs=pl.BlockSpec((BM, D), lambda m: (m, 0)),
        out_shape=jax.ShapeDtypeStruct((S, D), F32),
    )(h, yy, yy, w12)


def kernel(hidden_states, ln1_g, ln1_b, q_w, q_b, k_w, k_b, v_w, v_b,
           o_w, o_b, ln2_g, ln2_b, gate_w, fc1_w, fc2_w):
    B, S, D = hidden_states.shape
    E = gate_w.shape[0]
    H = 16
    HD = D // H
    x = hidden_states.reshape(S, D)

    wqkv = jnp.concatenate([q_w.T, k_w.T, v_w.T], axis=1)
    bqkv = jnp.concatenate([q_b, k_b, v_b]).reshape(1, 3 * D)
    qkv = _qkv(x, wqkv, bqkv, ln1_g.reshape(1, D), ln1_b.reshape(1, D))

    def heads(t):
        return t.reshape(S, H, HD).transpose(1, 0, 2)

    q3 = heads(qkv[:, :D])
    k3 = heads(qkv[:, D:2 * D])
    v3 = heads(qkv[:, 2 * D:])
    ctx = _attn(q3, k3, v3).transpose(1, 0, 2).reshape(S, D)

    gate_pad = jnp.zeros((EP, D), F32).at[:E].set(gate_w)
    h, h2, lg, e12, w12 = _oproj(ctx, o_w.T, o_b.reshape(1, D), x,
                                 ln2_g.reshape(1, D), ln2_b.reshape(1, D),
                                 gate_pad, E)

    pos, cnt, eflat = _dispatch_pos(e12, S)
    dest, bexp = _dispatch_dest(pos, eflat, cnt, E)
    bexp_s = bexp.reshape(EP)[:GMAX + 1]
    tok = jnp.arange(2 * S, dtype=jnp.int32) % S
    tokhl = jnp.stack([tok // 64, tok % 64, jnp.ones_like(tok)],
                      axis=1).astype(F32)
    tsrc = _invert(dest, tokhl, S)

    xs = _sc_dispatch(h2, tsrc.reshape(-1))
    Y = _moe_grouped(xs, fc1_w.astype(jnp.bfloat16),
                     fc2_w.astype(jnp.bfloat16), bexp_s)
    yy = _sc_combine_gather(Y, dest.reshape(-1))
    out = _combine(h, yy, w12)
    return (out.reshape(B, S, D), lg[:, :E])
